# baseline (speedup 1.0000x reference)
import functools

import jax
import jax.numpy as jnp
from jax import lax
from jax.experimental import pallas as pl
from jax.experimental.pallas import tpu as pltpu
from jax.experimental.pallas import tpu_sc as plsc

_D = 768
_LANES = 16
_NC = 2
_NS = 16
_NW = _NC * _NS


@functools.lru_cache(maxsize=None)
def _emb_kernel(n_rows: int, rows_pw: int, chunk: int, nbuf: int):
    mesh = plsc.VectorSubcoreMesh(core_axis_name="c", subcore_axis_name="s")
    n_chunks = rows_pw // chunk
    n_slices = _D // _LANES
    assert n_chunks % nbuf == 0

    @functools.partial(
        pl.kernel, mesh=mesh,
        out_type=jax.ShapeDtypeStruct((n_rows, _D), jnp.float32),
        scratch_types=[
            pltpu.VMEM((rows_pw,), jnp.int32),
            pltpu.VMEM((rows_pw,), jnp.int32),
            pltpu.VMEM((rows_pw + _LANES,), jnp.int32),
            pltpu.VMEM((2, _D), jnp.float32),
        ] + [pltpu.VMEM((chunk, _D), jnp.float32)] * (3 * nbuf)
          + [pltpu.SemaphoreType.DMA] * (2 * nbuf),
    )
    def body(*refs):
        (iw_hbm, ip_hbm, it_hbm, wt_hbm, pt_hbm, tt_hbm, out_hbm,
         iw_v, ip_v, it_v, tt_v) = refs[:11]
        bufs = refs[11:11 + 3 * nbuf]
        w_v = bufs[0::3]
        p_v = bufs[1::3]
        r_v = bufs[2::3]
        sems = refs[11 + 3 * nbuf:]
        gsem = sems[:nbuf]
        osem = sems[nbuf:]
        sid = lax.axis_index("s")
        wid = sid * _NC + lax.axis_index("c")
        base = wid * rows_pw
        pltpu.sync_copy(iw_hbm.at[pl.ds(base, rows_pw)], iw_v)
        pltpu.sync_copy(ip_hbm.at[pl.ds(base, rows_pw)], ip_v)
        pltpu.sync_copy(it_hbm.at[pl.ds(base, rows_pw)],
                        it_v.at[pl.ds(0, rows_pw)])
        pltpu.sync_copy(tt_hbm, tt_v)

        def fire_gathers(k, b):
            off = k * chunk
            pltpu.async_copy(wt_hbm.at[iw_v.at[pl.ds(off, chunk)]],
                             w_v[b], gsem[b])
            pltpu.async_copy(pt_hbm.at[ip_v.at[pl.ds(off, chunk)]],
                             p_v[b], gsem[b])

        def wait_gathers(k, b):
            off = k * chunk
            pltpu.make_async_copy(wt_hbm.at[iw_v.at[pl.ds(off, chunk)]],
                                  w_v[b], gsem[b]).wait()
            pltpu.make_async_copy(pt_hbm.at[ip_v.at[pl.ds(off, chunk)]],
                                  p_v[b], gsem[b]).wait()

        def wait_out(k, b):
            off = k * chunk
            pltpu.make_async_copy(r_v[b], out_hbm.at[pl.ds(base + off, chunk)],
                                  osem[b]).wait()

        for b in range(nbuf):
            fire_gathers(b, b)

        def do_group(g, carry):
            for b in range(nbuf):
                k = g * nbuf + b
                wait_gathers(k, b)

                @pl.when(g >= 1)
                def _():
                    wait_out(k - nbuf, b)

                tg = it_v[pl.ds(k * chunk, _LANES)].astype(jnp.float32)
                facs = []
                for r in range(chunk):
                    facs.append(jnp.full((_LANES,), tg[r], jnp.float32))

                def do_slice(j, carry2):
                    s = pl.ds(j * _LANES, _LANES)
                    t0 = tt_v[0, s]
                    dt = tt_v[1, s] - t0
                    for r in range(chunk):
                        r_v[b][r, s] = (w_v[b][r, s] + p_v[b][r, s]
                                        + (t0 + facs[r] * dt))
                    return carry2

                lax.fori_loop(0, n_slices, do_slice, 0)
                pltpu.async_copy(r_v[b], out_hbm.at[pl.ds(base + k * chunk, chunk)],
                                 osem[b])

                @pl.when(k + nbuf < n_chunks)
                def _():
                    fire_gathers(k + nbuf, b)
            return carry

        lax.fori_loop(0, n_chunks // nbuf, do_group, 0)
        for b in range(nbuf):
            wait_out(n_chunks - nbuf + b, b)

    return body


def kernel(input_ids, position_ids, token_type_ids, word_embeddings,
           position_embeddings, token_type_embeddings):
    b, s = input_ids.shape
    n_rows = b * s
    iw = input_ids.reshape(n_rows).astype(jnp.int32)
    ip = position_ids.reshape(n_rows).astype(jnp.int32)
    it = token_type_ids.reshape(n_rows).astype(jnp.int32)
    rows_pw = n_rows // _NW
    assert token_type_embeddings.shape[0] == 2, \
        "kernel specialized for a 2-row token-type table"
    k = _emb_kernel(n_rows, rows_pw, chunk=8, nbuf=4)
    out = k(iw, ip, it, word_embeddings, position_embeddings,
            token_type_embeddings)
    return out.reshape(b, s, _D)
